# Initial kernel scaffold; baseline (speedup 1.0000x reference)
import jax
import jax.numpy as jnp
from jax.experimental import pallas as pl


def kernel(x, edge_index, batch_index, W_l0, b_l0, W_r0, W_l1, b_l1, W_r1, W_l2, b_l2, W_r2):
    raise NotImplementedError("write your pallas kernel here")


# speedup vs baseline: 3.2707x; 3.2707x over previous
import functools

import jax
import jax.numpy as jnp
from jax import lax
from jax.experimental import pallas as pl
from jax.experimental.pallas import tpu as pltpu
from jax.experimental.pallas import tpu_sc as plsc

N = 10000
E = 160000
B = 64

NC = 2
NS = 16
K = 128
EP = 163840
G = EP // K
GPT = G // NS
NPT = 624
NA = N + 8

_MESH = dict(core_axis_name="c", subcore_axis_name="s", num_cores=NC,
             num_subcores=NS)


def _agg_body(C, with_deg, *refs):
    if with_deg:
        (h_hbm, src_hbm, dst_hbm, z128_hbm, zn_hbm,
         out_hbm, deg_hbm, src_buf, dst_buf, rows, ones_v, deg_tmp, acc,
         deg_acc, sem) = refs
    else:
        (h_hbm, src_hbm, dst_hbm, z128_hbm,
         out_hbm, src_buf, dst_buf, rows, acc, sem) = refs
    ci = lax.axis_index("c")
    si = lax.axis_index("s")
    CL = C // NC

    row0 = si * GPT
    pltpu.sync_copy(src_hbm.at[pl.ds(row0, GPT)], src_buf)
    pltpu.sync_copy(dst_hbm.at[pl.ds(row0, GPT)], dst_buf)

    if with_deg:
        for k in range(K // 16):
            ones_v[pl.ds(16 * k, 16)] = jnp.ones((16,), jnp.float32)

    for cl in range(CL):
        c = ci * CL + cl
        pltpu.sync_copy(z128_hbm.at[pl.ds(si * NPT, NPT)],
                        acc.at[pl.ds(si * NPT, NPT)])

        @pl.when(si == NS - 1)
        def _():
            pltpu.sync_copy(z128_hbm.at[pl.ds(NS * NPT, N - NS * NPT)],
                            acc.at[pl.ds(NS * NPT, N - NS * NPT)])
        if with_deg and cl == 0:
            @pl.when(jnp.logical_and(ci == 0, si < 5))
            def _():
                pltpu.sync_copy(zn_hbm.at[pl.ds(si * 2000, 2000)], deg_tmp)
                pltpu.sync_copy(deg_tmp, deg_acc.at[pl.ds(si * 2000, 2000)])
        plsc.subcore_barrier()

        do_deg = with_deg and cl == 0

        @pl.loop(0, GPT)
        def _(g):
            pltpu.async_copy(h_hbm.at[c].at[src_buf.at[g]], rows, sem).wait()
            pltpu.sync_copy(rows, acc.at[dst_buf.at[g]], add=True)
            if do_deg:
                @pl.when(ci == 0)
                def _():
                    pltpu.sync_copy(ones_v, deg_acc.at[dst_buf.at[g]],
                                    add=True)

        plsc.subcore_barrier()
        pltpu.sync_copy(acc.at[pl.ds(si * NPT, NPT)],
                        out_hbm.at[c].at[pl.ds(si * NPT, NPT)])

        @pl.when(si == NS - 1)
        def _():
            pltpu.sync_copy(acc.at[pl.ds(NS * NPT, N - NS * NPT)],
                            out_hbm.at[c].at[pl.ds(NS * NPT, N - NS * NPT)])
        if do_deg:
            @pl.when(jnp.logical_and(ci == 0, si < 5))
            def _():
                pltpu.sync_copy(deg_acc.at[pl.ds(si * 2000, 2000)], deg_tmp)
                pltpu.sync_copy(deg_tmp, deg_hbm.at[pl.ds(si * 2000, 2000)])
        if cl + 1 < CL:
            plsc.subcore_barrier()


def _make_agg(C, with_deg):
    out_type = [jax.ShapeDtypeStruct((C, N, 128), jnp.float32)]
    scratch = [
        pltpu.VMEM((GPT, K), jnp.int32),
        pltpu.VMEM((GPT, K), jnp.int32),
        pltpu.VMEM((K, 128), jnp.float32),
    ]
    if with_deg:
        out_type.append(jax.ShapeDtypeStruct((N,), jnp.float32))
        scratch.append(pltpu.VMEM((K,), jnp.float32))
        scratch.append(pltpu.VMEM((2000,), jnp.float32))
    scratch.append(pltpu.VMEM_SHARED((NA, 128), jnp.float32))
    if with_deg:
        scratch.append(pltpu.VMEM_SHARED((NA,), jnp.float32))
    scratch.append(pltpu.SemaphoreType.DMA)
    return pl.kernel(
        functools.partial(_agg_body, C, with_deg),
        out_type=out_type,
        mesh=plsc.VectorSubcoreMesh(**_MESH),
        scratch_types=scratch,
    )


def _tc1_body(a_ref, d_ref, x_ref, wl_ref, b_ref, wr_ref, o_ref):
    a = jnp.concatenate([a_ref[0], a_ref[1]], axis=1)
    rdeg = (1.0 / jnp.maximum(d_ref[0, 0, :], 1.0))[:, None]
    z = jnp.dot(a * rdeg, wl_ref[...], preferred_element_type=jnp.float32)
    z = z + b_ref[...] + jnp.dot(x_ref[...], wr_ref[...],
                                 preferred_element_type=jnp.float32)
    z = jnp.maximum(z, 0.0)
    for cc in range(4):
        o_ref[cc] = z[:, 128 * cc:128 * (cc + 1)]


def _tc2_body(a_ref, d_ref, h_ref, wl_ref, b_ref, wr_ref, wl2_ref,
              h2_ref, p2_ref):
    a = jnp.concatenate([a_ref[i] for i in range(4)], axis=1)
    h = jnp.concatenate([h_ref[i] for i in range(4)], axis=1)
    rdeg = (1.0 / jnp.maximum(d_ref[0, 0, :], 1.0))[:, None]
    z = jnp.dot(a * rdeg, wl_ref[...], preferred_element_type=jnp.float32)
    z = z + b_ref[...] + jnp.dot(h, wr_ref[...],
                                 preferred_element_type=jnp.float32)
    z = jnp.maximum(z, 0.0)
    h2_ref[...] = z
    p2 = jnp.dot(z, wl2_ref[...], preferred_element_type=jnp.float32)
    for cc in range(2):
        p2_ref[cc] = p2[:, 128 * cc:128 * (cc + 1)]


def _tc3_body(a_ref, d_ref, h2_ref, wr_ref, b_ref, bi_ref, o_ref,
              pool_acc, cnt_acc):
    i = pl.program_id(0)
    ni = pl.num_programs(0)

    @pl.when(i == 0)
    def _():
        pool_acc[...] = jnp.zeros_like(pool_acc)
        cnt_acc[...] = jnp.zeros_like(cnt_acc)

    a = jnp.concatenate([a_ref[0], a_ref[1]], axis=1)
    rdeg = (1.0 / jnp.maximum(d_ref[0, 0, :], 1.0))[:, None]
    h3 = a * rdeg + b_ref[...] + jnp.dot(h2_ref[...], wr_ref[...],
                                         preferred_element_type=jnp.float32)
    bi = bi_ref[0, 0, :]
    mb = h3.shape[0]
    onehot = (bi[None, :] == lax.broadcasted_iota(jnp.int32, (B, mb), 0)
              ).astype(jnp.float32)
    pool_acc[...] += jnp.dot(onehot, h3, preferred_element_type=jnp.float32)
    cnt_acc[...] += jnp.dot(onehot, jnp.ones((mb, 128), jnp.float32),
                            preferred_element_type=jnp.float32)

    @pl.when(i == ni - 1)
    def _():
        o_ref[...] = pool_acc[...] / jnp.maximum(cnt_acc[:, :1], 1.0)


_MB = 1000
_GN = N // _MB


def _tc1(agg0, deg3, x, Wl, b, Wr):
    return pl.pallas_call(
        _tc1_body,
        grid=(_GN,),
        in_specs=[
            pl.BlockSpec((2, _MB, 128), lambda i: (0, i, 0)),
            pl.BlockSpec((1, 1, _MB), lambda i: (i, 0, 0)),
            pl.BlockSpec((_MB, 256), lambda i: (i, 0)),
            pl.BlockSpec((256, 512), lambda i: (0, 0)),
            pl.BlockSpec((1, 512), lambda i: (0, 0)),
            pl.BlockSpec((256, 512), lambda i: (0, 0)),
        ],
        out_specs=pl.BlockSpec((4, _MB, 128), lambda i: (0, i, 0)),
        out_shape=jax.ShapeDtypeStruct((4, N, 128), jnp.float32),
    )(agg0, deg3, x, Wl, b, Wr)


def _tc2(agg1, deg3, h1c, Wl, b, Wr, Wl2):
    return pl.pallas_call(
        _tc2_body,
        grid=(_GN,),
        in_specs=[
            pl.BlockSpec((4, _MB, 128), lambda i: (0, i, 0)),
            pl.BlockSpec((1, 1, _MB), lambda i: (i, 0, 0)),
            pl.BlockSpec((4, _MB, 128), lambda i: (0, i, 0)),
            pl.BlockSpec((512, 512), lambda i: (0, 0)),
            pl.BlockSpec((1, 512), lambda i: (0, 0)),
            pl.BlockSpec((512, 512), lambda i: (0, 0)),
            pl.BlockSpec((512, 256), lambda i: (0, 0)),
        ],
        out_specs=(
            pl.BlockSpec((_MB, 512), lambda i: (i, 0)),
            pl.BlockSpec((2, _MB, 128), lambda i: (0, i, 0)),
        ),
        out_shape=(
            jax.ShapeDtypeStruct((N, 512), jnp.float32),
            jax.ShapeDtypeStruct((2, N, 128), jnp.float32),
        ),
    )(agg1, deg3, h1c, Wl, b, Wr, Wl2)


def _tc3(agg2, deg3, h2, Wr, b, bi3):
    return pl.pallas_call(
        _tc3_body,
        grid=(_GN,),
        in_specs=[
            pl.BlockSpec((2, _MB, 128), lambda i: (0, i, 0)),
            pl.BlockSpec((1, 1, _MB), lambda i: (i, 0, 0)),
            pl.BlockSpec((_MB, 512), lambda i: (i, 0)),
            pl.BlockSpec((512, 256), lambda i: (0, 0)),
            pl.BlockSpec((1, 256), lambda i: (0, 0)),
            pl.BlockSpec((1, 1, _MB), lambda i: (i, 0, 0)),
        ],
        out_specs=pl.BlockSpec((B, 256), lambda i: (0, 0)),
        out_shape=jax.ShapeDtypeStruct((B, 256), jnp.float32),
        scratch_shapes=[
            pltpu.VMEM((B, 256), jnp.float32),
            pltpu.VMEM((B, 128), jnp.float32),
        ],
    )(agg2, deg3, h2, Wr, b, bi3)


def kernel(x, edge_index, batch_index, W_l0, b_l0, W_r0, W_l1, b_l1, W_r1,
           W_l2, b_l2, W_r2):
    pad = EP - E
    src2d = jnp.concatenate(
        [edge_index[0], jnp.zeros((pad,), jnp.int32)]).reshape(G, K)
    dst2d = jnp.concatenate(
        [edge_index[1], jnp.full((pad,), N, jnp.int32)]).reshape(G, K)
    x_ch = x.reshape(N, 2, 128).transpose(1, 0, 2)
    z128 = jnp.zeros((N, 128), jnp.float32)
    zn = jnp.zeros((N,), jnp.float32)

    agg0, deg = _make_agg(2, True)(x_ch, src2d, dst2d, z128, zn)
    deg3 = deg.reshape(_GN, 1, _MB)
    h1c = _tc1(agg0, deg3, x, W_l0, b_l0.reshape(1, -1), W_r0)
    agg1, = _make_agg(4, False)(h1c, src2d, dst2d, z128)
    h2, p2c = _tc2(agg1, deg3, h1c, W_l1, b_l1.reshape(1, -1), W_r1, W_l2)
    agg2, = _make_agg(2, False)(p2c, src2d, dst2d, z128)
    return _tc3(agg2, deg3, h2, W_r2, b_l2.reshape(1, -1),
                batch_index.reshape(_GN, 1, _MB))


# speedup vs baseline: 3.5590x; 1.0881x over previous
import functools

import jax
import jax.numpy as jnp
from jax import lax
from jax.experimental import pallas as pl
from jax.experimental.pallas import tpu as pltpu
from jax.experimental.pallas import tpu_sc as plsc

N = 10000
E = 160000
B = 64

NC = 2
NS = 16
K = 128
EP = 163840
G = EP // K
GPT = G // NS
NPT = 624
NA = N + 8
RS = 16

_MESH = dict(core_axis_name="c", subcore_axis_name="s", num_cores=NC,
             num_subcores=NS)


def _agg_body(C, with_deg, *refs):
    if with_deg:
        (h_hbm, src_hbm, dst_hbm, z128_hbm, zn_hbm,
         out_hbm, deg_hbm, src_st, dst_st, rows, ones_v, deg_tmp, acc,
         deg_acc, gs0, gs1, ss0, ss1) = refs
    else:
        (h_hbm, src_hbm, dst_hbm, z128_hbm,
         out_hbm, src_st, dst_st, rows, acc, gs0, gs1, ss0, ss1) = refs
    gsem = (gs0, gs1)
    ssem = (ss0, ss1)
    ci = lax.axis_index("c")
    si = lax.axis_index("s")
    CL = C // NC
    row0 = si * GPT

    if with_deg:
        for k in range(K // 16):
            ones_v[pl.ds(16 * k, 16)] = jnp.ones((16,), jnp.float32)

    for cl in range(CL):
        c = ci * CL + cl
        pltpu.sync_copy(z128_hbm.at[pl.ds(si * NPT, NPT)],
                        acc.at[pl.ds(si * NPT, NPT)])

        @pl.when(si == NS - 1)
        def _():
            pltpu.sync_copy(z128_hbm.at[pl.ds(NS * NPT, N - NS * NPT)],
                            acc.at[pl.ds(NS * NPT, N - NS * NPT)])
        if with_deg and cl == 0:
            @pl.when(jnp.logical_and(ci == 0, si < 5))
            def _():
                pltpu.sync_copy(zn_hbm.at[pl.ds(si * 2000, 2000)], deg_tmp)
                pltpu.sync_copy(deg_tmp, deg_acc.at[pl.ds(si * 2000, 2000)])
        plsc.subcore_barrier()

        do_deg = with_deg and cl == 0

        def _gather(r, b):
            pltpu.async_copy(h_hbm.at[c].at[src_st.at[r]], rows.at[b],
                             gsem[b])

        def _wait_gather(b):
            pltpu.make_async_copy(h_hbm.at[c].at[src_st.at[0]], rows.at[b],
                                  gsem[b]).wait()

        def _scatter(r, b):
            pltpu.async_copy(rows.at[b], acc.at[dst_st.at[r]], ssem[b],
                             add=True)

        def _wait_scatter(b):
            pltpu.make_async_copy(rows.at[b], acc.at[dst_st.at[0]],
                                  ssem[b]).wait()

        def _do_deg(r):
            if do_deg:
                @pl.when(ci == 0)
                def _():
                    pltpu.sync_copy(ones_v, deg_acc.at[dst_st.at[r]],
                                    add=True)

        @pl.loop(0, GPT, step=2)
        def _(g0):
            r0 = lax.rem(g0, RS)

            @pl.when(r0 == 0)
            def _():
                @pl.when(g0 > 0)
                def _():
                    _wait_scatter(0)
                    _wait_scatter(1)
                base = pl.multiple_of(row0 + g0, 8)
                pltpu.sync_copy(src_hbm.at[pl.ds(base, RS)], src_st)
                pltpu.sync_copy(dst_hbm.at[pl.ds(base, RS)], dst_st)
                _gather(0, 0)
                _gather(1, 1)

            _wait_gather(0)
            _scatter(r0, 0)
            _do_deg(r0)
            _wait_gather(1)
            _scatter(r0 + 1, 1)
            _do_deg(r0 + 1)

            @pl.when(r0 + 2 < RS)
            def _():
                _wait_scatter(0)
                _gather(r0 + 2, 0)

            @pl.when(r0 + 3 < RS)
            def _():
                _wait_scatter(1)
                _gather(r0 + 3, 1)

        _wait_scatter(0)
        _wait_scatter(1)

        plsc.subcore_barrier()
        pltpu.sync_copy(acc.at[pl.ds(si * NPT, NPT)],
                        out_hbm.at[c].at[pl.ds(si * NPT, NPT)])

        @pl.when(si == NS - 1)
        def _():
            pltpu.sync_copy(acc.at[pl.ds(NS * NPT, N - NS * NPT)],
                            out_hbm.at[c].at[pl.ds(NS * NPT, N - NS * NPT)])
        if do_deg:
            @pl.when(jnp.logical_and(ci == 0, si < 5))
            def _():
                pltpu.sync_copy(deg_acc.at[pl.ds(si * 2000, 2000)], deg_tmp)
                pltpu.sync_copy(deg_tmp, deg_hbm.at[pl.ds(si * 2000, 2000)])
        if cl + 1 < CL:
            plsc.subcore_barrier()


def _make_agg(C, with_deg):
    out_type = [jax.ShapeDtypeStruct((C, N, 128), jnp.float32)]
    scratch = [
        pltpu.VMEM((RS, K), jnp.int32),
        pltpu.VMEM((RS, K), jnp.int32),
        pltpu.VMEM((2, K, 128), jnp.float32),
    ]
    if with_deg:
        out_type.append(jax.ShapeDtypeStruct((N,), jnp.float32))
        scratch.append(pltpu.VMEM((K,), jnp.float32))
        scratch.append(pltpu.VMEM((2000,), jnp.float32))
    scratch.append(pltpu.VMEM_SHARED((NA, 128), jnp.float32))
    if with_deg:
        scratch.append(pltpu.VMEM_SHARED((NA,), jnp.float32))
    scratch.extend([pltpu.SemaphoreType.DMA] * 4)
    return pl.kernel(
        functools.partial(_agg_body, C, with_deg),
        out_type=out_type,
        mesh=plsc.VectorSubcoreMesh(**_MESH),
        scratch_types=scratch,
    )


def _tc1_body(a_ref, d_ref, x_ref, wl_ref, b_ref, wr_ref, o_ref):
    a = jnp.concatenate([a_ref[0], a_ref[1]], axis=1)
    rdeg = (1.0 / jnp.maximum(d_ref[0, 0, :], 1.0))[:, None]
    z = jnp.dot(a * rdeg, wl_ref[...], preferred_element_type=jnp.float32)
    z = z + b_ref[...] + jnp.dot(x_ref[...], wr_ref[...],
                                 preferred_element_type=jnp.float32)
    z = jnp.maximum(z, 0.0)
    for cc in range(4):
        o_ref[cc] = z[:, 128 * cc:128 * (cc + 1)]


def _tc2_body(a_ref, d_ref, h_ref, wl_ref, b_ref, wr_ref, wl2_ref,
              h2_ref, p2_ref):
    a = jnp.concatenate([a_ref[i] for i in range(4)], axis=1)
    h = jnp.concatenate([h_ref[i] for i in range(4)], axis=1)
    rdeg = (1.0 / jnp.maximum(d_ref[0, 0, :], 1.0))[:, None]
    z = jnp.dot(a * rdeg, wl_ref[...], preferred_element_type=jnp.float32)
    z = z + b_ref[...] + jnp.dot(h, wr_ref[...],
                                 preferred_element_type=jnp.float32)
    z = jnp.maximum(z, 0.0)
    h2_ref[...] = z
    p2 = jnp.dot(z, wl2_ref[...], preferred_element_type=jnp.float32)
    for cc in range(2):
        p2_ref[cc] = p2[:, 128 * cc:128 * (cc + 1)]


def _tc3_body(a_ref, d_ref, h2_ref, wr_ref, b_ref, bi_ref, o_ref,
              pool_acc, cnt_acc):
    i = pl.program_id(0)
    ni = pl.num_programs(0)

    @pl.when(i == 0)
    def _():
        pool_acc[...] = jnp.zeros_like(pool_acc)
        cnt_acc[...] = jnp.zeros_like(cnt_acc)

    a = jnp.concatenate([a_ref[0], a_ref[1]], axis=1)
    rdeg = (1.0 / jnp.maximum(d_ref[0, 0, :], 1.0))[:, None]
    h3 = a * rdeg + b_ref[...] + jnp.dot(h2_ref[...], wr_ref[...],
                                         preferred_element_type=jnp.float32)
    bi = bi_ref[0, 0, :]
    mb = h3.shape[0]
    onehot = (bi[None, :] == lax.broadcasted_iota(jnp.int32, (B, mb), 0)
              ).astype(jnp.float32)
    pool_acc[...] += jnp.dot(onehot, h3, preferred_element_type=jnp.float32)
    cnt_acc[...] += jnp.dot(onehot, jnp.ones((mb, 128), jnp.float32),
                            preferred_element_type=jnp.float32)

    @pl.when(i == ni - 1)
    def _():
        o_ref[...] = pool_acc[...] / jnp.maximum(cnt_acc[:, :1], 1.0)


_MB = 1000
_GN = N // _MB


def _tc1(agg0, deg3, x, Wl, b, Wr):
    return pl.pallas_call(
        _tc1_body,
        grid=(_GN,),
        in_specs=[
            pl.BlockSpec((2, _MB, 128), lambda i: (0, i, 0)),
            pl.BlockSpec((1, 1, _MB), lambda i: (i, 0, 0)),
            pl.BlockSpec((_MB, 256), lambda i: (i, 0)),
            pl.BlockSpec((256, 512), lambda i: (0, 0)),
            pl.BlockSpec((1, 512), lambda i: (0, 0)),
            pl.BlockSpec((256, 512), lambda i: (0, 0)),
        ],
        out_specs=pl.BlockSpec((4, _MB, 128), lambda i: (0, i, 0)),
        out_shape=jax.ShapeDtypeStruct((4, N, 128), jnp.float32),
    )(agg0, deg3, x, Wl, b, Wr)


def _tc2(agg1, deg3, h1c, Wl, b, Wr, Wl2):
    return pl.pallas_call(
        _tc2_body,
        grid=(_GN,),
        in_specs=[
            pl.BlockSpec((4, _MB, 128), lambda i: (0, i, 0)),
            pl.BlockSpec((1, 1, _MB), lambda i: (i, 0, 0)),
            pl.BlockSpec((4, _MB, 128), lambda i: (0, i, 0)),
            pl.BlockSpec((512, 512), lambda i: (0, 0)),
            pl.BlockSpec((1, 512), lambda i: (0, 0)),
            pl.BlockSpec((512, 512), lambda i: (0, 0)),
            pl.BlockSpec((512, 256), lambda i: (0, 0)),
        ],
        out_specs=(
            pl.BlockSpec((_MB, 512), lambda i: (i, 0)),
            pl.BlockSpec((2, _MB, 128), lambda i: (0, i, 0)),
        ),
        out_shape=(
            jax.ShapeDtypeStruct((N, 512), jnp.float32),
            jax.ShapeDtypeStruct((2, N, 128), jnp.float32),
        ),
    )(agg1, deg3, h1c, Wl, b, Wr, Wl2)


def _tc3(agg2, deg3, h2, Wr, b, bi3):
    return pl.pallas_call(
        _tc3_body,
        grid=(_GN,),
        in_specs=[
            pl.BlockSpec((2, _MB, 128), lambda i: (0, i, 0)),
            pl.BlockSpec((1, 1, _MB), lambda i: (i, 0, 0)),
            pl.BlockSpec((_MB, 512), lambda i: (i, 0)),
            pl.BlockSpec((512, 256), lambda i: (0, 0)),
            pl.BlockSpec((1, 256), lambda i: (0, 0)),
            pl.BlockSpec((1, 1, _MB), lambda i: (i, 0, 0)),
        ],
        out_specs=pl.BlockSpec((B, 256), lambda i: (0, 0)),
        out_shape=jax.ShapeDtypeStruct((B, 256), jnp.float32),
        scratch_shapes=[
            pltpu.VMEM((B, 256), jnp.float32),
            pltpu.VMEM((B, 128), jnp.float32),
        ],
    )(agg2, deg3, h2, Wr, b, bi3)


def kernel(x, edge_index, batch_index, W_l0, b_l0, W_r0, W_l1, b_l1, W_r1,
           W_l2, b_l2, W_r2):
    pad = EP - E
    src2d = jnp.concatenate(
        [edge_index[0], jnp.zeros((pad,), jnp.int32)]).reshape(G, K)
    dst2d = jnp.concatenate(
        [edge_index[1], jnp.full((pad,), N, jnp.int32)]).reshape(G, K)
    x_ch = x.reshape(N, 2, 128).transpose(1, 0, 2)
    z128 = jnp.zeros((N, 128), jnp.float32)
    zn = jnp.zeros((N,), jnp.float32)

    agg0, deg = _make_agg(2, True)(x_ch, src2d, dst2d, z128, zn)
    deg3 = deg.reshape(_GN, 1, _MB)
    h1c = _tc1(agg0, deg3, x, W_l0, b_l0.reshape(1, -1), W_r0)
    agg1, = _make_agg(4, False)(h1c, src2d, dst2d, z128)
    h2, p2c = _tc2(agg1, deg3, h1c, W_l1, b_l1.reshape(1, -1), W_r1, W_l2)
    agg2, = _make_agg(2, False)(p2c, src2d, dst2d, z128)
    return _tc3(agg2, deg3, h2, W_r2, b_l2.reshape(1, -1),
                batch_index.reshape(_GN, 1, _MB))


# speedup vs baseline: 3.6158x; 1.0159x over previous
import functools

import jax
import jax.numpy as jnp
from jax import lax
from jax.experimental import pallas as pl
from jax.experimental.pallas import tpu as pltpu
from jax.experimental.pallas import tpu_sc as plsc

N = 10000
E = 160000
B = 64

NC = 2
NS = 16
K = 128
EP = 163840
G = EP // K
GPT = G // NS
NPT = 624
NA = N + 8
RS = 40

_MESH = dict(core_axis_name="c", subcore_axis_name="s", num_cores=NC,
             num_subcores=NS)


def _agg_body(C, with_deg, *refs):
    if with_deg:
        (h_hbm, src_hbm, dst_hbm, z128_hbm, zn_hbm,
         out_hbm, deg_hbm, src_st, dst_st, rows, ones_v, deg_tmp, acc,
         deg_acc, gs0, gs1, ss0, ss1) = refs
    else:
        (h_hbm, src_hbm, dst_hbm, z128_hbm,
         out_hbm, src_st, dst_st, rows, acc, gs0, gs1, ss0, ss1) = refs
    gsem = (gs0, gs1)
    ssem = (ss0, ss1)
    ci = lax.axis_index("c")
    si = lax.axis_index("s")
    CL = C // NC
    row0 = si * GPT

    if with_deg:
        for k in range(K // 16):
            ones_v[pl.ds(16 * k, 16)] = jnp.ones((16,), jnp.float32)

    for cl in range(CL):
        c = ci * CL + cl
        pltpu.sync_copy(z128_hbm.at[pl.ds(si * NPT, NPT)],
                        acc.at[pl.ds(si * NPT, NPT)])

        @pl.when(si == NS - 1)
        def _():
            pltpu.sync_copy(z128_hbm.at[pl.ds(NS * NPT, N - NS * NPT)],
                            acc.at[pl.ds(NS * NPT, N - NS * NPT)])
        if with_deg and cl == 0:
            @pl.when(jnp.logical_and(ci == 0, si < 5))
            def _():
                pltpu.sync_copy(zn_hbm.at[pl.ds(si * 2000, 2000)], deg_tmp)
                pltpu.sync_copy(deg_tmp, deg_acc.at[pl.ds(si * 2000, 2000)])
        plsc.subcore_barrier()

        do_deg = with_deg and cl == 0

        def _gather(r, b):
            pltpu.async_copy(h_hbm.at[c].at[src_st.at[r]], rows.at[b],
                             gsem[b])

        def _wait_gather(b):
            pltpu.make_async_copy(h_hbm.at[c].at[src_st.at[0]], rows.at[b],
                                  gsem[b]).wait()

        def _scatter(r, b):
            pltpu.async_copy(rows.at[b], acc.at[dst_st.at[r]], ssem[b],
                             add=True)

        def _wait_scatter(b):
            pltpu.make_async_copy(rows.at[b], acc.at[dst_st.at[0]],
                                  ssem[b]).wait()

        def _do_deg(r):
            if do_deg:
                @pl.when(ci == 0)
                def _():
                    pltpu.sync_copy(ones_v, deg_acc.at[dst_st.at[r]],
                                    add=True)

        @pl.loop(0, GPT, step=2)
        def _(g0):
            r0 = lax.rem(g0, RS)

            @pl.when(r0 == 0)
            def _():
                @pl.when(g0 > 0)
                def _():
                    _wait_scatter(0)
                    _wait_scatter(1)
                base = pl.multiple_of(row0 + g0, 8)
                pltpu.sync_copy(src_hbm.at[pl.ds(base, RS)], src_st)
                pltpu.sync_copy(dst_hbm.at[pl.ds(base, RS)], dst_st)
                _gather(0, 0)
                _gather(1, 1)

            _wait_gather(0)
            _scatter(r0, 0)
            _do_deg(r0)
            _wait_gather(1)
            _scatter(r0 + 1, 1)
            _do_deg(r0 + 1)

            @pl.when(r0 + 2 < RS)
            def _():
                _wait_scatter(0)
                _gather(r0 + 2, 0)

            @pl.when(r0 + 3 < RS)
            def _():
                _wait_scatter(1)
                _gather(r0 + 3, 1)

        _wait_scatter(0)
        _wait_scatter(1)

        plsc.subcore_barrier()
        pltpu.sync_copy(acc.at[pl.ds(si * NPT, NPT)],
                        out_hbm.at[c].at[pl.ds(si * NPT, NPT)])

        @pl.when(si == NS - 1)
        def _():
            pltpu.sync_copy(acc.at[pl.ds(NS * NPT, N - NS * NPT)],
                            out_hbm.at[c].at[pl.ds(NS * NPT, N - NS * NPT)])
        if do_deg:
            @pl.when(jnp.logical_and(ci == 0, si < 5))
            def _():
                pltpu.sync_copy(deg_acc.at[pl.ds(si * 2000, 2000)], deg_tmp)
                pltpu.sync_copy(deg_tmp, deg_hbm.at[pl.ds(si * 2000, 2000)])
        if cl + 1 < CL:
            plsc.subcore_barrier()


def _make_agg(C, with_deg):
    out_type = [jax.ShapeDtypeStruct((C, N, 128), jnp.float32)]
    scratch = [
        pltpu.VMEM((RS, K), jnp.int32),
        pltpu.VMEM((RS, K), jnp.int32),
        pltpu.VMEM((2, K, 128), jnp.float32),
    ]
    if with_deg:
        out_type.append(jax.ShapeDtypeStruct((N,), jnp.float32))
        scratch.append(pltpu.VMEM((K,), jnp.float32))
        scratch.append(pltpu.VMEM((2000,), jnp.float32))
    scratch.append(pltpu.VMEM_SHARED((NA, 128), jnp.float32))
    if with_deg:
        scratch.append(pltpu.VMEM_SHARED((NA,), jnp.float32))
    scratch.extend([pltpu.SemaphoreType.DMA] * 4)
    return pl.kernel(
        functools.partial(_agg_body, C, with_deg),
        out_type=out_type,
        mesh=plsc.VectorSubcoreMesh(**_MESH),
        scratch_types=scratch,
    )


def _tc1_body(a_ref, d_ref, x_ref, wl_ref, b_ref, wr_ref, o_ref):
    a = jnp.concatenate([a_ref[0], a_ref[1]], axis=1)
    rdeg = (1.0 / jnp.maximum(d_ref[0, 0, :], 1.0))[:, None]
    z = jnp.dot(a * rdeg, wl_ref[...], preferred_element_type=jnp.float32)
    z = z + b_ref[...] + jnp.dot(x_ref[...], wr_ref[...],
                                 preferred_element_type=jnp.float32)
    z = jnp.maximum(z, 0.0)
    for cc in range(4):
        o_ref[cc] = z[:, 128 * cc:128 * (cc + 1)]


def _tc2_body(a_ref, d_ref, h_ref, wl_ref, b_ref, wr_ref, wl2_ref,
              h2_ref, p2_ref):
    a = jnp.concatenate([a_ref[i] for i in range(4)], axis=1)
    h = jnp.concatenate([h_ref[i] for i in range(4)], axis=1)
    rdeg = (1.0 / jnp.maximum(d_ref[0, 0, :], 1.0))[:, None]
    z = jnp.dot(a * rdeg, wl_ref[...], preferred_element_type=jnp.float32)
    z = z + b_ref[...] + jnp.dot(h, wr_ref[...],
                                 preferred_element_type=jnp.float32)
    z = jnp.maximum(z, 0.0)
    h2_ref[...] = z
    p2 = jnp.dot(z, wl2_ref[...], preferred_element_type=jnp.float32)
    for cc in range(2):
        p2_ref[cc] = p2[:, 128 * cc:128 * (cc + 1)]


def _tc3_body(a_ref, d_ref, h2_ref, wr_ref, b_ref, bi_ref, o_ref,
              pool_acc, cnt_acc):
    i = pl.program_id(0)
    ni = pl.num_programs(0)

    @pl.when(i == 0)
    def _():
        pool_acc[...] = jnp.zeros_like(pool_acc)
        cnt_acc[...] = jnp.zeros_like(cnt_acc)

    a = jnp.concatenate([a_ref[0], a_ref[1]], axis=1)
    rdeg = (1.0 / jnp.maximum(d_ref[0, 0, :], 1.0))[:, None]
    h3 = a * rdeg + b_ref[...] + jnp.dot(h2_ref[...], wr_ref[...],
                                         preferred_element_type=jnp.float32)
    bi = bi_ref[0, 0, :]
    mb = h3.shape[0]
    onehot = (bi[None, :] == lax.broadcasted_iota(jnp.int32, (B, mb), 0)
              ).astype(jnp.float32)
    pool_acc[...] += jnp.dot(onehot, h3, preferred_element_type=jnp.float32)
    cnt_acc[...] += jnp.dot(onehot, jnp.ones((mb, 128), jnp.float32),
                            preferred_element_type=jnp.float32)

    @pl.when(i == ni - 1)
    def _():
        o_ref[...] = pool_acc[...] / jnp.maximum(cnt_acc[:, :1], 1.0)


_MB = 1000
_GN = N // _MB


def _tc1(agg0, deg3, x, Wl, b, Wr):
    return pl.pallas_call(
        _tc1_body,
        grid=(_GN,),
        in_specs=[
            pl.BlockSpec((2, _MB, 128), lambda i: (0, i, 0)),
            pl.BlockSpec((1, 1, _MB), lambda i: (i, 0, 0)),
            pl.BlockSpec((_MB, 256), lambda i: (i, 0)),
            pl.BlockSpec((256, 512), lambda i: (0, 0)),
            pl.BlockSpec((1, 512), lambda i: (0, 0)),
            pl.BlockSpec((256, 512), lambda i: (0, 0)),
        ],
        out_specs=pl.BlockSpec((4, _MB, 128), lambda i: (0, i, 0)),
        out_shape=jax.ShapeDtypeStruct((4, N, 128), jnp.float32),
    )(agg0, deg3, x, Wl, b, Wr)


def _tc2(agg1, deg3, h1c, Wl, b, Wr, Wl2):
    return pl.pallas_call(
        _tc2_body,
        grid=(_GN,),
        in_specs=[
            pl.BlockSpec((4, _MB, 128), lambda i: (0, i, 0)),
            pl.BlockSpec((1, 1, _MB), lambda i: (i, 0, 0)),
            pl.BlockSpec((4, _MB, 128), lambda i: (0, i, 0)),
            pl.BlockSpec((512, 512), lambda i: (0, 0)),
            pl.BlockSpec((1, 512), lambda i: (0, 0)),
            pl.BlockSpec((512, 512), lambda i: (0, 0)),
            pl.BlockSpec((512, 256), lambda i: (0, 0)),
        ],
        out_specs=(
            pl.BlockSpec((_MB, 512), lambda i: (i, 0)),
            pl.BlockSpec((2, _MB, 128), lambda i: (0, i, 0)),
        ),
        out_shape=(
            jax.ShapeDtypeStruct((N, 512), jnp.float32),
            jax.ShapeDtypeStruct((2, N, 128), jnp.float32),
        ),
    )(agg1, deg3, h1c, Wl, b, Wr, Wl2)


def _tc3(agg2, deg3, h2, Wr, b, bi3):
    return pl.pallas_call(
        _tc3_body,
        grid=(_GN,),
        in_specs=[
            pl.BlockSpec((2, _MB, 128), lambda i: (0, i, 0)),
            pl.BlockSpec((1, 1, _MB), lambda i: (i, 0, 0)),
            pl.BlockSpec((_MB, 512), lambda i: (i, 0)),
            pl.BlockSpec((512, 256), lambda i: (0, 0)),
            pl.BlockSpec((1, 256), lambda i: (0, 0)),
            pl.BlockSpec((1, 1, _MB), lambda i: (i, 0, 0)),
        ],
        out_specs=pl.BlockSpec((B, 256), lambda i: (0, 0)),
        out_shape=jax.ShapeDtypeStruct((B, 256), jnp.float32),
        scratch_shapes=[
            pltpu.VMEM((B, 256), jnp.float32),
            pltpu.VMEM((B, 128), jnp.float32),
        ],
    )(agg2, deg3, h2, Wr, b, bi3)


def kernel(x, edge_index, batch_index, W_l0, b_l0, W_r0, W_l1, b_l1, W_r1,
           W_l2, b_l2, W_r2):
    pad = EP - E
    src2d = jnp.concatenate(
        [edge_index[0], jnp.zeros((pad,), jnp.int32)]).reshape(G, K)
    dst2d = jnp.concatenate(
        [edge_index[1], jnp.full((pad,), N, jnp.int32)]).reshape(G, K)
    x_ch = x.reshape(N, 2, 128).transpose(1, 0, 2)
    z128 = jnp.zeros((N, 128), jnp.float32)
    zn = jnp.zeros((N,), jnp.float32)

    agg0, deg = _make_agg(2, True)(x_ch, src2d, dst2d, z128, zn)
    deg3 = deg.reshape(_GN, 1, _MB)
    h1c = _tc1(agg0, deg3, x, W_l0, b_l0.reshape(1, -1), W_r0)
    agg1, = _make_agg(4, False)(h1c, src2d, dst2d, z128)
    h2, p2c = _tc2(agg1, deg3, h1c, W_l1, b_l1.reshape(1, -1), W_r1, W_l2)
    agg2, = _make_agg(2, False)(p2c, src2d, dst2d, z128)
    return _tc3(agg2, deg3, h2, W_r2, b_l2.reshape(1, -1),
                batch_index.reshape(_GN, 1, _MB))
